# baseline (speedup 1.0000x reference)
import functools

import jax
import jax.numpy as jnp
from jax import lax
from jax.experimental import pallas as pl
from jax.experimental.pallas import tpu as pltpu
from jax.experimental.pallas import tpu_sc as plsc

N = 10000
D = 128
NC = 2
NS = 16
NW = NC * NS
CH = 128
NCH = 79
SLOTS = NW * NCH * CH
NCH0 = 110
NCH1 = 47
DSEG = 22
SLAB = 25
SEG0 = [(22 * s, 22) for s in range(5)]
SEG1 = [(0, 22), (22, 22), (44, 3)]
SROWS = 640
NPAD = NS * SROWS
SCHUNKS = [(k * CH, CH) for k in range(SROWS // CH)]

_mesh = plsc.VectorSubcoreMesh(core_axis_name="c", subcore_axis_name="s")


def _deg_body(dstI_hbm, z1_hbm, out_hbm, dst_v, ones_v, dbuf, deg_sh):
    cid = lax.axis_index("c")
    sid = lax.axis_index("s")
    t = cid * NS + sid
    soff = pl.multiple_of(sid * SROWS, 8)
    for j in range(CH // 16):
        ones_v[pl.ds(j * 16, 16)] = jnp.ones((16,), jnp.float32)
    pltpu.sync_copy(z1_hbm, dbuf)
    pltpu.sync_copy(dbuf, deg_sh.at[pl.ds(soff, SROWS)])
    pltpu.sync_copy(dstI_hbm.at[t], dst_v)
    plsc.subcore_barrier()

    def body(c, carry):
        pltpu.sync_copy(ones_v, deg_sh.at[dst_v.at[c]], add=True)
        return carry

    lax.fori_loop(0, NCH, body, 0)
    plsc.subcore_barrier()
    ooff = pl.multiple_of(cid * NPAD + sid * SROWS, 8)
    pltpu.sync_copy(deg_sh.at[pl.ds(soff, SROWS)], dbuf)
    pltpu.sync_copy(dbuf, out_hbm.at[pl.ds(ooff, SROWS)])


_deg_call = pl.kernel(
    _deg_body,
    out_type=jax.ShapeDtypeStruct((NC * NPAD,), jnp.float32),
    mesh=_mesh,
    scratch_types=[
        pltpu.VMEM((NCH, CH), jnp.int32),
        pltpu.VMEM((CH,), jnp.float32),
        pltpu.VMEM((SROWS,), jnp.float32),
        pltpu.VMEM_SHARED((NPAD,), jnp.float32),
    ],
)


def _prime(srcS, sid, srcA, bufA, bufB, semA, semB, h_hbm):
    pltpu.sync_copy(srcS.at[0, sid], srcA)
    pltpu.async_copy(h_hbm.at[srcA.at[0]], bufA, semA)
    pltpu.async_copy(h_hbm.at[srcA.at[1]], bufB, semB)


def _run_pipeline(segs, nch, srcS, dstS, sid, srcA, srcB, dst_v,
                  bufA, bufB, semA, semB, agg_sh, h_hbm):
    sb = [srcA, srcB]
    for s, (base, cnt) in enumerate(segs):
        sv = sb[s % 2]
        pltpu.sync_copy(dstS.at[s, sid], dst_v)
        lim = min(nch - 1 - base, SLAB - 1)

        def body(i, carry, sv=sv, lim=lim):
            r0 = i * 2
            pltpu.make_async_copy(h_hbm.at[sv.at[0]], bufA, semA).wait()
            pltpu.sync_copy(bufA, agg_sh.at[dst_v.at[r0]], add=True)
            pltpu.async_copy(h_hbm.at[sv.at[jnp.minimum(r0 + 2, lim)]],
                             bufA, semA)
            pltpu.make_async_copy(h_hbm.at[sv.at[0]], bufB, semB).wait()
            pltpu.sync_copy(bufB, agg_sh.at[dst_v.at[r0 + 1]], add=True)
            pltpu.async_copy(h_hbm.at[sv.at[jnp.minimum(r0 + 3, lim)]],
                             bufB, semB)
            return carry

        lax.fori_loop(0, cnt // 2, body, 0)
        if cnt % 2:
            pltpu.make_async_copy(h_hbm.at[sv.at[0]], bufA, semA).wait()
            pltpu.sync_copy(bufA, agg_sh.at[dst_v.at[cnt - 1]], add=True)
        if s + 1 < len(segs):
            pltpu.sync_copy(srcS.at[s + 1, sid], sb[(s + 1) % 2])
    if segs[-1][1] % 2 == 0:
        pltpu.make_async_copy(h_hbm.at[srcA.at[0]], bufA, semA).wait()
    pltpu.make_async_copy(h_hbm.at[srcA.at[0]], bufB, semB).wait()


def _agg_body(h_hbm, srcS0_hbm, dstS0_hbm, srcS1_hbm, dstS1_hbm, z2_hbm,
              out_hbm, srcA, srcB, dst_v, bufA, bufB, semA, semB, semZ,
              agg_sh):
    cid = lax.axis_index("c")
    sid = lax.axis_index("s")

    @pl.when(cid == 0)
    def _():
        _prime(srcS0_hbm, sid, srcA, bufA, bufB, semA, semB, h_hbm)

    @pl.when(cid == 1)
    def _():
        _prime(srcS1_hbm, sid, srcA, bufA, bufB, semA, semB, h_hbm)

    for k0, kn in SCHUNKS:
        koff = pl.multiple_of(sid * SROWS + k0, 8)
        pltpu.async_copy(z2_hbm, agg_sh.at[pl.ds(koff, kn)], semZ)
    for k0, kn in SCHUNKS:
        koff = pl.multiple_of(sid * SROWS + k0, 8)
        pltpu.make_async_copy(z2_hbm, agg_sh.at[pl.ds(koff, kn)], semZ).wait()
    plsc.subcore_barrier()

    @pl.when(cid == 0)
    def _():
        _run_pipeline(SEG0, NCH0, srcS0_hbm, dstS0_hbm, sid, srcA, srcB,
                      dst_v, bufA, bufB, semA, semB, agg_sh, h_hbm)

    @pl.when(cid == 1)
    def _():
        _run_pipeline(SEG1, NCH1, srcS1_hbm, dstS1_hbm, sid, srcA, srcB,
                      dst_v, bufA, bufB, semA, semB, agg_sh, h_hbm)

    plsc.subcore_barrier()
    for k, (k0, kn) in enumerate(SCHUNKS):
        koff = pl.multiple_of(sid * SROWS + k0, 8)
        buf = bufA if k % 2 == 0 else bufB
        sem = semA if k % 2 == 0 else semB
        if k >= 2:
            pltpu.make_async_copy(buf, out_hbm.at[cid, pl.ds(0, kn)],
                                  sem).wait()
        pltpu.sync_copy(agg_sh.at[pl.ds(koff, kn)], buf)
        pltpu.async_copy(buf, out_hbm.at[cid, pl.ds(koff, kn)], sem)
    pltpu.make_async_copy(bufA, out_hbm.at[cid, pl.ds(0, CH)], semA).wait()
    pltpu.make_async_copy(bufB, out_hbm.at[cid, pl.ds(0, CH)], semB).wait()


_agg_call = pl.kernel(
    _agg_body,
    out_type=jax.ShapeDtypeStruct((NC, NPAD, D), jnp.float32),
    mesh=_mesh,
    scratch_types=[
        pltpu.VMEM((SLAB, CH), jnp.int32),
        pltpu.VMEM((SLAB, CH), jnp.int32),
        pltpu.VMEM((DSEG, CH), jnp.int32),
        pltpu.VMEM((CH, D), jnp.float32),
        pltpu.VMEM((CH, D), jnp.float32),
        pltpu.SemaphoreType.DMA,
        pltpu.SemaphoreType.DMA,
        pltpu.SemaphoreType.DMA,
        pltpu.VMEM_SHARED((NPAD, D), jnp.float32),
    ],
)


def _tca_body(dd_ref, x_ref, w_ref, h1s_ref, dinv_ref):
    deg = dd_ref[:, 0:1] + dd_ref[:, 1:2] + 1.0
    dinv = lax.rsqrt(deg)
    h = jnp.dot(x_ref[...], w_ref[...], preferred_element_type=jnp.float32)
    h1s_ref[...] = h * dinv
    dinv_ref[...] = dinv


def _tca(dd, x, w):
    return pl.pallas_call(
        _tca_body,
        out_shape=(jax.ShapeDtypeStruct((N, D), jnp.float32),
                   jax.ShapeDtypeStruct((N, 1), jnp.float32)),
    )(dd, x, w)


def _tcb_body(aggp_ref, hs_ref, dinv_ref, b_ref, g_ref, be_ref, w_ref,
              hbn_ref, hs2_ref):
    dinv = dinv_ref[...]
    a = aggp_ref[0, :N, :] + aggp_ref[1, :N, :] + hs_ref[...]
    pre = a * dinv + b_ref[...]
    mu = jnp.mean(pre, axis=0, keepdims=True)
    var = jnp.mean((pre - mu) ** 2, axis=0, keepdims=True)
    hbn = g_ref[...] * (pre - mu) / jnp.sqrt(var + 1e-5) + be_ref[...]
    hbn = jnp.maximum(hbn, 0.0)
    hbn_ref[...] = hbn
    hs2_ref[...] = jnp.dot(hbn, w_ref[...],
                           preferred_element_type=jnp.float32) * dinv


def _tcb(aggp, hs, dinv, b, g, be, w):
    return pl.pallas_call(
        _tcb_body,
        out_shape=(jax.ShapeDtypeStruct((N, D), jnp.float32),
                   jax.ShapeDtypeStruct((N, D), jnp.float32)),
    )(aggp, hs, dinv, b, g, be, w)


def _tcd_body(aggp_ref, hs_ref, dinv_ref, b_ref, out_ref):
    a = aggp_ref[0, :N, :] + aggp_ref[1, :N, :] + hs_ref[...]
    out_ref[...] = a * dinv_ref[...] + b_ref[...]


def _tcd(aggp, hs, dinv, b):
    return pl.pallas_call(
        _tcd_body,
        out_shape=jax.ShapeDtypeStruct((N, D), jnp.float32),
    )(aggp, hs, dinv, b)


def kernel(x, edge_index, W1, b1, g1, be1, W2, b2, g2, be2, W3, b3):
    src = edge_index[0].astype(jnp.int32)
    dst = edge_index[1].astype(jnp.int32)
    e = src.shape[0]
    pad = SLOTS - e
    dstI = jnp.concatenate([dst, jnp.full((pad,), N, jnp.int32)])
    dstI = dstI.reshape(NW, NCH, CH)

    def slabs(sv, dv, ntile_ch, nseg):
        need = 22 * (nseg - 1) + SLAB
        sp = jnp.pad(sv, ((0, 0), (0, need - ntile_ch), (0, 0)))
        dp = jnp.pad(dv, ((0, 0), (0, need - ntile_ch), (0, 0)),
                     constant_values=N)
        sS = jnp.stack([sp[:, 22 * s:22 * s + SLAB] for s in range(nseg)])
        dS = jnp.stack([dp[:, 22 * s:22 * s + DSEG] for s in range(nseg)])
        return sS, dS

    e0 = NS * NCH0 * CH
    cap1 = NS * NCH1 * CH
    pad1 = e0 + cap1 - e
    src_p = jnp.concatenate([src, jnp.zeros((pad1,), jnp.int32)])
    dst_p = jnp.concatenate([dst, jnp.full((pad1,), N, jnp.int32)])
    srcS0, dstS0 = slabs(src_p[:e0].reshape(NS, NCH0, CH),
                         dst_p[:e0].reshape(NS, NCH0, CH), NCH0, len(SEG0))
    srcS1, dstS1 = slabs(src_p[e0:].reshape(NS, NCH1, CH),
                         dst_p[e0:].reshape(NS, NCH1, CH), NCH1, len(SEG1))
    z1 = jnp.zeros((SROWS,), jnp.float32)
    z2 = jnp.zeros((CH, D), jnp.float32)

    degp = _deg_call(dstI, z1).reshape(NC, NPAD)
    dd = degp[:, :N].T
    h1s, dinv = _tca(dd, x, W1)

    aggp1 = _agg_call(h1s, srcS0, dstS0, srcS1, dstS1, z2)
    _, h2s = _tcb(aggp1, h1s, dinv, b1, g1, be1, W2)

    aggp2 = _agg_call(h2s, srcS0, dstS0, srcS1, dstS1, z2)
    h_out, h3s = _tcb(aggp2, h2s, dinv, b2, g2, be2, W3)

    aggp3 = _agg_call(h3s, srcS0, dstS0, srcS1, dstS1, z2)
    out = _tcd(aggp3, h3s, dinv, b3)
    return (h_out, out)


# speedup vs baseline: 1.0537x; 1.0537x over previous
import functools

import jax
import jax.numpy as jnp
from jax import lax
from jax.experimental import pallas as pl
from jax.experimental.pallas import tpu as pltpu
from jax.experimental.pallas import tpu_sc as plsc

N = 10000
D = 128
NC = 2
NS = 16
NW = NC * NS
CH = 128
NCH = 79
SLOTS = NW * NCH * CH
NCH0 = 110
NCH1 = 47
DSEG = 22
SLAB = 25
SEG0 = [(22 * s, 22) for s in range(5)]
SEG1 = [(0, 22), (22, 22), (44, 3)]
SROWS = 640
NPAD = NS * SROWS
SCHUNKS = [(k * CH, CH) for k in range(SROWS // CH)]

_mesh = plsc.VectorSubcoreMesh(core_axis_name="c", subcore_axis_name="s")


def _deg_body(dstI_hbm, z1_hbm, out_hbm, dst_v, ones_v, dbuf, deg_sh):
    cid = lax.axis_index("c")
    sid = lax.axis_index("s")
    t = cid * NS + sid
    soff = pl.multiple_of(sid * SROWS, 8)
    for j in range(CH // 16):
        ones_v[pl.ds(j * 16, 16)] = jnp.ones((16,), jnp.float32)
    pltpu.sync_copy(z1_hbm, dbuf)
    pltpu.sync_copy(dbuf, deg_sh.at[pl.ds(soff, SROWS)])
    pltpu.sync_copy(dstI_hbm.at[t], dst_v)
    plsc.subcore_barrier()

    def body(c, carry):
        pltpu.sync_copy(ones_v, deg_sh.at[dst_v.at[c]], add=True)
        return carry

    lax.fori_loop(0, NCH, body, 0)
    plsc.subcore_barrier()
    ooff = pl.multiple_of(cid * NPAD + sid * SROWS, 8)
    pltpu.sync_copy(deg_sh.at[pl.ds(soff, SROWS)], dbuf)
    pltpu.sync_copy(dbuf, out_hbm.at[pl.ds(ooff, SROWS)])


_deg_call = pl.kernel(
    _deg_body,
    out_type=jax.ShapeDtypeStruct((NC * NPAD,), jnp.float32),
    mesh=_mesh,
    scratch_types=[
        pltpu.VMEM((NCH, CH), jnp.int32),
        pltpu.VMEM((CH,), jnp.float32),
        pltpu.VMEM((SROWS,), jnp.float32),
        pltpu.VMEM_SHARED((NPAD,), jnp.float32),
    ],
)


def _run_pipeline(segs, nch, srcS, dstS, sid, srcA, srcB, dst_v,
                  bufA, bufB, semA, semB, agg_sh, h_hbm):
    sb = [srcA, srcB]
    pltpu.sync_copy(srcS.at[0, sid], srcA)
    pltpu.async_copy(h_hbm.at[srcA.at[0]], bufA, semA)
    pltpu.async_copy(h_hbm.at[srcA.at[1]], bufB, semB)
    for s, (base, cnt) in enumerate(segs):
        sv = sb[s % 2]
        pltpu.sync_copy(dstS.at[s, sid], dst_v)
        lim = min(nch - 1 - base, SLAB - 1)

        def body(i, carry, sv=sv, lim=lim):
            r0 = i * 2
            pltpu.make_async_copy(h_hbm.at[sv.at[0]], bufA, semA).wait()
            pltpu.sync_copy(bufA, agg_sh.at[dst_v.at[r0]], add=True)
            pltpu.async_copy(h_hbm.at[sv.at[jnp.minimum(r0 + 2, lim)]],
                             bufA, semA)
            pltpu.make_async_copy(h_hbm.at[sv.at[0]], bufB, semB).wait()
            pltpu.sync_copy(bufB, agg_sh.at[dst_v.at[r0 + 1]], add=True)
            pltpu.async_copy(h_hbm.at[sv.at[jnp.minimum(r0 + 3, lim)]],
                             bufB, semB)
            return carry

        lax.fori_loop(0, cnt // 2, body, 0)
        if cnt % 2:
            pltpu.make_async_copy(h_hbm.at[sv.at[0]], bufA, semA).wait()
            pltpu.sync_copy(bufA, agg_sh.at[dst_v.at[cnt - 1]], add=True)
        if s + 1 < len(segs):
            pltpu.sync_copy(srcS.at[s + 1, sid], sb[(s + 1) % 2])
    if segs[-1][1] % 2 == 0:
        pltpu.make_async_copy(h_hbm.at[srcA.at[0]], bufA, semA).wait()
    pltpu.make_async_copy(h_hbm.at[srcA.at[0]], bufB, semB).wait()


def _agg_body(h_hbm, srcS0_hbm, dstS0_hbm, srcS1_hbm, dstS1_hbm, z2_hbm,
              out_hbm, srcA, srcB, dst_v, bufA, bufB, semA, semB, agg_sh):
    cid = lax.axis_index("c")
    sid = lax.axis_index("s")
    pltpu.sync_copy(z2_hbm, bufA)
    for k0, kn in SCHUNKS:
        koff = pl.multiple_of(sid * SROWS + k0, 8)
        pltpu.sync_copy(bufA.at[pl.ds(0, kn)], agg_sh.at[pl.ds(koff, kn)])
    plsc.subcore_barrier()

    @pl.when(cid == 0)
    def _():
        _run_pipeline(SEG0, NCH0, srcS0_hbm, dstS0_hbm, sid, srcA, srcB,
                      dst_v, bufA, bufB, semA, semB, agg_sh, h_hbm)

    @pl.when(cid == 1)
    def _():
        _run_pipeline(SEG1, NCH1, srcS1_hbm, dstS1_hbm, sid, srcA, srcB,
                      dst_v, bufA, bufB, semA, semB, agg_sh, h_hbm)

    plsc.subcore_barrier()
    for k, (k0, kn) in enumerate(SCHUNKS):
        koff = pl.multiple_of(sid * SROWS + k0, 8)
        buf = bufA if k % 2 == 0 else bufB
        sem = semA if k % 2 == 0 else semB
        if k >= 2:
            pltpu.make_async_copy(buf, out_hbm.at[cid, pl.ds(0, kn)],
                                  sem).wait()
        pltpu.sync_copy(agg_sh.at[pl.ds(koff, kn)], buf)
        pltpu.async_copy(buf, out_hbm.at[cid, pl.ds(koff, kn)], sem)
    pltpu.make_async_copy(bufA, out_hbm.at[cid, pl.ds(0, CH)], semA).wait()
    pltpu.make_async_copy(bufB, out_hbm.at[cid, pl.ds(0, CH)], semB).wait()


_agg_call = pl.kernel(
    _agg_body,
    out_type=jax.ShapeDtypeStruct((NC, NPAD, D), jnp.float32),
    mesh=_mesh,
    scratch_types=[
        pltpu.VMEM((SLAB, CH), jnp.int32),
        pltpu.VMEM((SLAB, CH), jnp.int32),
        pltpu.VMEM((DSEG, CH), jnp.int32),
        pltpu.VMEM((CH, D), jnp.float32),
        pltpu.VMEM((CH, D), jnp.float32),
        pltpu.SemaphoreType.DMA,
        pltpu.SemaphoreType.DMA,
        pltpu.VMEM_SHARED((NPAD, D), jnp.float32),
    ],
)


def _tca_body(dd_ref, x_ref, w_ref, h1s_ref, dinv_ref):
    deg = dd_ref[:, 0:1] + dd_ref[:, 1:2] + 1.0
    dinv = lax.rsqrt(deg)
    h = jnp.dot(x_ref[...], w_ref[...], preferred_element_type=jnp.float32)
    h1s_ref[...] = h * dinv
    dinv_ref[...] = dinv


def _tca(dd, x, w):
    return pl.pallas_call(
        _tca_body,
        out_shape=(jax.ShapeDtypeStruct((N, D), jnp.float32),
                   jax.ShapeDtypeStruct((N, 1), jnp.float32)),
    )(dd, x, w)


def _tcb_body(aggp_ref, hs_ref, dinv_ref, b_ref, g_ref, be_ref, w_ref,
              hbn_ref, hs2_ref):
    dinv = dinv_ref[...]
    a = aggp_ref[0, :N, :] + aggp_ref[1, :N, :] + hs_ref[...]
    pre = a * dinv + b_ref[...]
    mu = jnp.mean(pre, axis=0, keepdims=True)
    var = jnp.mean((pre - mu) ** 2, axis=0, keepdims=True)
    hbn = g_ref[...] * (pre - mu) / jnp.sqrt(var + 1e-5) + be_ref[...]
    hbn = jnp.maximum(hbn, 0.0)
    hbn_ref[...] = hbn
    hs2_ref[...] = jnp.dot(hbn, w_ref[...],
                           preferred_element_type=jnp.float32) * dinv


def _tcb(aggp, hs, dinv, b, g, be, w):
    return pl.pallas_call(
        _tcb_body,
        out_shape=(jax.ShapeDtypeStruct((N, D), jnp.float32),
                   jax.ShapeDtypeStruct((N, D), jnp.float32)),
    )(aggp, hs, dinv, b, g, be, w)


def _tcd_body(aggp_ref, hs_ref, dinv_ref, b_ref, out_ref):
    a = aggp_ref[0, :N, :] + aggp_ref[1, :N, :] + hs_ref[...]
    out_ref[...] = a * dinv_ref[...] + b_ref[...]


def _tcd(aggp, hs, dinv, b):
    return pl.pallas_call(
        _tcd_body,
        out_shape=jax.ShapeDtypeStruct((N, D), jnp.float32),
    )(aggp, hs, dinv, b)


def kernel(x, edge_index, W1, b1, g1, be1, W2, b2, g2, be2, W3, b3):
    src = edge_index[0].astype(jnp.int32)
    dst = edge_index[1].astype(jnp.int32)
    e = src.shape[0]
    pad = SLOTS - e
    dstI = jnp.concatenate([dst, jnp.full((pad,), N, jnp.int32)])
    dstI = dstI.reshape(NW, NCH, CH)

    def slabs(sv, dv, ntile_ch, nseg):
        need = 22 * (nseg - 1) + SLAB
        sp = jnp.pad(sv, ((0, 0), (0, need - ntile_ch), (0, 0)))
        dp = jnp.pad(dv, ((0, 0), (0, need - ntile_ch), (0, 0)),
                     constant_values=N)
        sS = jnp.stack([sp[:, 22 * s:22 * s + SLAB] for s in range(nseg)])
        dS = jnp.stack([dp[:, 22 * s:22 * s + DSEG] for s in range(nseg)])
        return sS, dS

    e0 = NS * NCH0 * CH
    cap1 = NS * NCH1 * CH
    pad1 = e0 + cap1 - e
    src_p = jnp.concatenate([src, jnp.zeros((pad1,), jnp.int32)])
    dst_p = jnp.concatenate([dst, jnp.full((pad1,), N, jnp.int32)])
    srcS0, dstS0 = slabs(src_p[:e0].reshape(NS, NCH0, CH),
                         dst_p[:e0].reshape(NS, NCH0, CH), NCH0, len(SEG0))
    srcS1, dstS1 = slabs(src_p[e0:].reshape(NS, NCH1, CH),
                         dst_p[e0:].reshape(NS, NCH1, CH), NCH1, len(SEG1))
    z1 = jnp.zeros((SROWS,), jnp.float32)
    z2 = jnp.zeros((CH, D), jnp.float32)

    degp = _deg_call(dstI, z1).reshape(NC, NPAD)
    dd = degp[:, :N].T
    h1s, dinv = _tca(dd, x, W1)

    aggp1 = _agg_call(h1s, srcS0, dstS0, srcS1, dstS1, z2)
    _, h2s = _tcb(aggp1, h1s, dinv, b1, g1, be1, W2)

    aggp2 = _agg_call(h2s, srcS0, dstS0, srcS1, dstS1, z2)
    h_out, h3s = _tcb(aggp2, h2s, dinv, b2, g2, be2, W3)

    aggp3 = _agg_call(h3s, srcS0, dstS0, srcS1, dstS1, z2)
    out = _tcd(aggp3, h3s, dinv, b3)
    return (h_out, out)


# speedup vs baseline: 1.0556x; 1.0018x over previous
import functools

import jax
import jax.numpy as jnp
from jax import lax
from jax.experimental import pallas as pl
from jax.experimental.pallas import tpu as pltpu
from jax.experimental.pallas import tpu_sc as plsc

N = 10000
D = 128
NC = 2
NS = 16
NW = NC * NS
CH = 128
NCH = 79
SLOTS = NW * NCH * CH
NCH0 = 110
NCH1 = 47
DSEG = 22
SLAB = 25
SEG0 = [(22 * s, 22) for s in range(5)]
SEG1 = [(0, 22), (22, 22), (44, 3)]
SROWS = 640
NPAD = NS * SROWS
SCHUNKS = [(k * CH, CH) for k in range(SROWS // CH)]

_mesh = plsc.VectorSubcoreMesh(core_axis_name="c", subcore_axis_name="s")


def _deg_body(dstI_hbm, z1_hbm, out_hbm, dst_v, ones_v, dbuf, deg_sh):
    cid = lax.axis_index("c")
    sid = lax.axis_index("s")
    t = cid * NS + sid
    soff = pl.multiple_of(sid * SROWS, 8)
    for j in range(CH // 16):
        ones_v[pl.ds(j * 16, 16)] = jnp.ones((16,), jnp.float32)
    pltpu.sync_copy(z1_hbm, dbuf)
    pltpu.sync_copy(dbuf, deg_sh.at[pl.ds(soff, SROWS)])
    pltpu.sync_copy(dstI_hbm.at[t], dst_v)
    plsc.subcore_barrier()

    def body(c, carry):
        pltpu.sync_copy(ones_v, deg_sh.at[dst_v.at[c]], add=True)
        return carry

    lax.fori_loop(0, NCH, body, 0)
    plsc.subcore_barrier()
    ooff = pl.multiple_of(cid * NPAD + sid * SROWS, 8)
    pltpu.sync_copy(deg_sh.at[pl.ds(soff, SROWS)], dbuf)
    pltpu.sync_copy(dbuf, out_hbm.at[pl.ds(ooff, SROWS)])


_deg_call = pl.kernel(
    _deg_body,
    out_type=jax.ShapeDtypeStruct((NC * NPAD,), jnp.float32),
    mesh=_mesh,
    scratch_types=[
        pltpu.VMEM((NCH, CH), jnp.int32),
        pltpu.VMEM((CH,), jnp.float32),
        pltpu.VMEM((SROWS,), jnp.float32),
        pltpu.VMEM_SHARED((NPAD,), jnp.float32),
    ],
)


def _run_pipeline(segs, nch, srcS, dstS, sid, srcA, srcB, dst_v,
                  bufA, bufB, semA, semB, agg_sh, h_hbm):
    sb = [srcA, srcB]
    pltpu.sync_copy(srcS.at[0, sid], srcA)
    pltpu.async_copy(h_hbm.at[srcA.at[0]], bufA, semA)
    pltpu.async_copy(h_hbm.at[srcA.at[1]], bufB, semB)
    for s, (base, cnt) in enumerate(segs):
        sv = sb[s % 2]
        pltpu.sync_copy(dstS.at[s, sid], dst_v)
        lim = min(nch - 1 - base, SLAB - 1)

        def body(i, carry, sv=sv, lim=lim):
            r0 = i * 2
            pltpu.make_async_copy(h_hbm.at[sv.at[0]], bufA, semA).wait()
            pltpu.sync_copy(bufA, agg_sh.at[dst_v.at[r0]], add=True)
            pltpu.async_copy(h_hbm.at[sv.at[jnp.minimum(r0 + 2, lim)]],
                             bufA, semA)
            pltpu.make_async_copy(h_hbm.at[sv.at[0]], bufB, semB).wait()
            pltpu.sync_copy(bufB, agg_sh.at[dst_v.at[r0 + 1]], add=True)
            pltpu.async_copy(h_hbm.at[sv.at[jnp.minimum(r0 + 3, lim)]],
                             bufB, semB)
            return carry

        lax.fori_loop(0, cnt // 2, body, 0)
        if cnt % 2:
            pltpu.make_async_copy(h_hbm.at[sv.at[0]], bufA, semA).wait()
            pltpu.sync_copy(bufA, agg_sh.at[dst_v.at[cnt - 1]], add=True)
        if s + 1 < len(segs):
            pltpu.sync_copy(srcS.at[s + 1, sid], sb[(s + 1) % 2])
    if segs[-1][1] % 2 == 0:
        pltpu.make_async_copy(h_hbm.at[srcA.at[0]], bufA, semA).wait()
    pltpu.make_async_copy(h_hbm.at[srcA.at[0]], bufB, semB).wait()


def _agg_body(h_hbm, srcS0_hbm, dstS0_hbm, srcS1_hbm, dstS1_hbm, z2_hbm,
              out_hbm, srcA, srcB, dst_v, bufA, bufB, semA, semB, agg_sh):
    cid = lax.axis_index("c")
    sid = lax.axis_index("s")
    pltpu.sync_copy(z2_hbm, bufA)
    for k0, kn in SCHUNKS:
        koff = pl.multiple_of(sid * SROWS + k0, 8)
        pltpu.async_copy(bufA.at[pl.ds(0, kn)], agg_sh.at[pl.ds(koff, kn)],
                         semA)
    for k0, kn in SCHUNKS:
        koff = pl.multiple_of(sid * SROWS + k0, 8)
        pltpu.make_async_copy(bufA.at[pl.ds(0, kn)],
                              agg_sh.at[pl.ds(koff, kn)], semA).wait()
    plsc.subcore_barrier()

    @pl.when(cid == 0)
    def _():
        _run_pipeline(SEG0, NCH0, srcS0_hbm, dstS0_hbm, sid, srcA, srcB,
                      dst_v, bufA, bufB, semA, semB, agg_sh, h_hbm)

    @pl.when(cid == 1)
    def _():
        _run_pipeline(SEG1, NCH1, srcS1_hbm, dstS1_hbm, sid, srcA, srcB,
                      dst_v, bufA, bufB, semA, semB, agg_sh, h_hbm)

    plsc.subcore_barrier()
    for k, (k0, kn) in enumerate(SCHUNKS):
        koff = pl.multiple_of(sid * SROWS + k0, 8)
        buf = bufA if k % 2 == 0 else bufB
        sem = semA if k % 2 == 0 else semB
        if k >= 2:
            pltpu.make_async_copy(buf, out_hbm.at[cid, pl.ds(0, kn)],
                                  sem).wait()
        pltpu.sync_copy(agg_sh.at[pl.ds(koff, kn)], buf)
        pltpu.async_copy(buf, out_hbm.at[cid, pl.ds(koff, kn)], sem)
    pltpu.make_async_copy(bufA, out_hbm.at[cid, pl.ds(0, CH)], semA).wait()
    pltpu.make_async_copy(bufB, out_hbm.at[cid, pl.ds(0, CH)], semB).wait()


_agg_call = pl.kernel(
    _agg_body,
    out_type=jax.ShapeDtypeStruct((NC, NPAD, D), jnp.float32),
    mesh=_mesh,
    scratch_types=[
        pltpu.VMEM((SLAB, CH), jnp.int32),
        pltpu.VMEM((SLAB, CH), jnp.int32),
        pltpu.VMEM((DSEG, CH), jnp.int32),
        pltpu.VMEM((CH, D), jnp.float32),
        pltpu.VMEM((CH, D), jnp.float32),
        pltpu.SemaphoreType.DMA,
        pltpu.SemaphoreType.DMA,
        pltpu.VMEM_SHARED((NPAD, D), jnp.float32),
    ],
)


def _tca_body(dd_ref, x_ref, w_ref, h1s_ref, dinv_ref):
    deg = dd_ref[:, 0:1] + dd_ref[:, 1:2] + 1.0
    dinv = lax.rsqrt(deg)
    h = jnp.dot(x_ref[...], w_ref[...], preferred_element_type=jnp.float32)
    h1s_ref[...] = h * dinv
    dinv_ref[...] = dinv


def _tca(dd, x, w):
    return pl.pallas_call(
        _tca_body,
        out_shape=(jax.ShapeDtypeStruct((N, D), jnp.float32),
                   jax.ShapeDtypeStruct((N, 1), jnp.float32)),
    )(dd, x, w)


def _tcb_body(aggp_ref, hs_ref, dinv_ref, b_ref, g_ref, be_ref, w_ref,
              hbn_ref, hs2_ref):
    dinv = dinv_ref[...]
    a = aggp_ref[0, :N, :] + aggp_ref[1, :N, :] + hs_ref[...]
    pre = a * dinv + b_ref[...]
    mu = jnp.mean(pre, axis=0, keepdims=True)
    var = jnp.mean((pre - mu) ** 2, axis=0, keepdims=True)
    hbn = g_ref[...] * (pre - mu) / jnp.sqrt(var + 1e-5) + be_ref[...]
    hbn = jnp.maximum(hbn, 0.0)
    hbn_ref[...] = hbn
    hs2_ref[...] = jnp.dot(hbn, w_ref[...],
                           preferred_element_type=jnp.float32) * dinv


def _tcb(aggp, hs, dinv, b, g, be, w):
    return pl.pallas_call(
        _tcb_body,
        out_shape=(jax.ShapeDtypeStruct((N, D), jnp.float32),
                   jax.ShapeDtypeStruct((N, D), jnp.float32)),
    )(aggp, hs, dinv, b, g, be, w)


def _tcd_body(aggp_ref, hs_ref, dinv_ref, b_ref, out_ref):
    a = aggp_ref[0, :N, :] + aggp_ref[1, :N, :] + hs_ref[...]
    out_ref[...] = a * dinv_ref[...] + b_ref[...]


def _tcd(aggp, hs, dinv, b):
    return pl.pallas_call(
        _tcd_body,
        out_shape=jax.ShapeDtypeStruct((N, D), jnp.float32),
    )(aggp, hs, dinv, b)


def kernel(x, edge_index, W1, b1, g1, be1, W2, b2, g2, be2, W3, b3):
    src = edge_index[0].astype(jnp.int32)
    dst = edge_index[1].astype(jnp.int32)
    e = src.shape[0]
    pad = SLOTS - e
    dstI = jnp.concatenate([dst, jnp.full((pad,), N, jnp.int32)])
    dstI = dstI.reshape(NW, NCH, CH)

    def slabs(sv, dv, ntile_ch, nseg):
        need = 22 * (nseg - 1) + SLAB
        sp = jnp.pad(sv, ((0, 0), (0, need - ntile_ch), (0, 0)))
        dp = jnp.pad(dv, ((0, 0), (0, need - ntile_ch), (0, 0)),
                     constant_values=N)
        sS = jnp.stack([sp[:, 22 * s:22 * s + SLAB] for s in range(nseg)])
        dS = jnp.stack([dp[:, 22 * s:22 * s + DSEG] for s in range(nseg)])
        return sS, dS

    e0 = NS * NCH0 * CH
    cap1 = NS * NCH1 * CH
    pad1 = e0 + cap1 - e
    src_p = jnp.concatenate([src, jnp.zeros((pad1,), jnp.int32)])
    dst_p = jnp.concatenate([dst, jnp.full((pad1,), N, jnp.int32)])
    srcS0, dstS0 = slabs(src_p[:e0].reshape(NS, NCH0, CH),
                         dst_p[:e0].reshape(NS, NCH0, CH), NCH0, len(SEG0))
    srcS1, dstS1 = slabs(src_p[e0:].reshape(NS, NCH1, CH),
                         dst_p[e0:].reshape(NS, NCH1, CH), NCH1, len(SEG1))
    z1 = jnp.zeros((SROWS,), jnp.float32)
    z2 = jnp.zeros((CH, D), jnp.float32)

    degp = _deg_call(dstI, z1).reshape(NC, NPAD)
    dd = degp[:, :N].T
    h1s, dinv = _tca(dd, x, W1)

    aggp1 = _agg_call(h1s, srcS0, dstS0, srcS1, dstS1, z2)
    _, h2s = _tcb(aggp1, h1s, dinv, b1, g1, be1, W2)

    aggp2 = _agg_call(h2s, srcS0, dstS0, srcS1, dstS1, z2)
    h_out, h3s = _tcb(aggp2, h2s, dinv, b2, g2, be2, W3)

    aggp3 = _agg_call(h3s, srcS0, dstS0, srcS1, dstS1, z2)
    out = _tcd(aggp3, h3s, dinv, b3)
    return (h_out, out)
